# Initial kernel scaffold; baseline (speedup 1.0000x reference)
import jax
import jax.numpy as jnp
from jax.experimental import pallas as pl


def kernel(x, edge_index, W_psi0, b_psi0, W_phi0, b_phi0, W_psi1, b_psi1, W_phi1, b_phi1, W_down, b_down):
    raise NotImplementedError("write your pallas kernel here")


# speedup vs baseline: 4.9922x; 4.9922x over previous
import jax
import jax.numpy as jnp
from jax import lax
from jax.experimental import pallas as pl
from jax.experimental.pallas import tpu as pltpu
from jax.experimental.pallas import tpu_sc as plsc

NC = 2
NS = 16
NW = NC * NS
LANES = 16

_PREC = jax.lax.Precision.HIGHEST


def _grid_consts(n, e):
    epw = e // NW
    c = 80
    nchunk = epw // c
    n_pad = -(-n // (NS * 128)) * (NS * 128)
    rpw = n_pad // NS
    assert epw * NW == e and c * nchunk == epw
    return epw, c, nchunk, n_pad, rpw


def _sc_mesh():
    return plsc.VectorSubcoreMesh(
        core_axis_name="c", subcore_axis_name="s",
        num_cores=NC, num_subcores=NS)


def _make_edge_kernel(n, d, e):
    epw, c, nchunk, n_pad, rpw = _grid_consts(n, e)
    assert d % LANES == 0
    dv = d // LANES

    scratch = [
        pltpu.VMEM((c,), jnp.int32),
        pltpu.VMEM((c,), jnp.int32),
        pltpu.VMEM((c, d), jnp.float32),
        pltpu.VMEM((c, d), jnp.float32),
        pltpu.VMEM_SHARED((n_pad, d), jnp.float32),
        pltpu.SemaphoreType.DMA,
        pltpu.SemaphoreType.DMA,
    ]

    def body(a_hbm, b_hbm, src_hbm, dst_hbm, z_hbm, out_hbm,
             idx_s, idx_d, rows_a, rows_b, acc, sem_a, sem_b):
        cid = lax.axis_index("c")
        sid = lax.axis_index("s")
        wid = cid * NS + sid
        rs = pl.ds(sid * rpw, rpw)

        pltpu.sync_copy(z_hbm.at[rs], acc.at[rs])
        plsc.subcore_barrier()

        base = wid * epw

        def chunk(k, carry):
            off = base + k * c
            pltpu.sync_copy(src_hbm.at[pl.ds(off, c)], idx_s)
            pltpu.sync_copy(dst_hbm.at[pl.ds(off, c)], idx_d)
            cp_a = pltpu.async_copy(a_hbm.at[idx_d], rows_a, sem_a)
            cp_b = pltpu.async_copy(b_hbm.at[idx_s], rows_b, sem_b)
            cp_a.wait()
            cp_b.wait()

            def row(r, rcarry):
                for j in range(dv):
                    sl = pl.ds(j * LANES, LANES)
                    rows_a[r, sl] = jnp.maximum(rows_a[r, sl] + rows_b[r, sl],
                                                0.0)
                return rcarry
            lax.fori_loop(0, c, row, 0)
            pltpu.sync_copy(rows_a, acc.at[idx_d], add=True)
            return carry
        lax.fori_loop(0, nchunk, chunk, 0)

        plsc.subcore_barrier()
        pltpu.sync_copy(acc.at[rs], out_hbm.at[cid, rs])

    return pl.kernel(body,
                     out_type=jax.ShapeDtypeStruct((NC, n_pad, d),
                                                   jnp.float32),
                     mesh=_sc_mesh(), scratch_types=scratch)


def _make_count_kernel(n, d, e):
    epw, c, nchunk, n_pad, rpw = _grid_consts(n, e)
    dv = d // LANES

    scratch = [
        pltpu.VMEM((c,), jnp.int32),
        pltpu.VMEM((c, d), jnp.float32),
        pltpu.VMEM_SHARED((n_pad, d), jnp.float32),
    ]

    def body(dst_hbm, z_hbm, cnt_hbm, idx_d, ones_v, cnt_acc):
        cid = lax.axis_index("c")
        sid = lax.axis_index("s")
        wid = cid * NS + sid
        one16 = jnp.ones((LANES,), jnp.float32)
        rs = pl.ds(sid * rpw, rpw)

        def orow(r, carry):
            for j in range(dv):
                ones_v[r, pl.ds(j * LANES, LANES)] = one16
            return carry
        lax.fori_loop(0, c, orow, 0)
        pltpu.sync_copy(z_hbm.at[rs], cnt_acc.at[rs])
        plsc.subcore_barrier()

        base = wid * epw

        def chunk(k, carry):
            pltpu.sync_copy(dst_hbm.at[pl.ds(base + k * c, c)], idx_d)
            pltpu.sync_copy(ones_v, cnt_acc.at[idx_d], add=True)
            return carry
        lax.fori_loop(0, nchunk, chunk, 0)

        plsc.subcore_barrier()
        pltpu.sync_copy(cnt_acc.at[rs], cnt_hbm.at[cid, rs])

    return pl.kernel(body,
                     out_type=jax.ShapeDtypeStruct((NC, n_pad, d),
                                                   jnp.float32),
                     mesh=_sc_mesh(), scratch_types=scratch)


def _dot(a, b):
    return jnp.dot(a, b, preferred_element_type=jnp.float32, precision=_PREC)


def _psi_pre(x, wl, wr, bp):
    n, d = x.shape
    bn = 2000
    grid = (n // bn,)

    def body(x_ref, wl_ref, wr_ref, bp_ref, a_ref, b_ref):
        xb = x_ref[...]
        a_ref[...] = _dot(xb, wl_ref[...]) + bp_ref[...]
        b_ref[...] = _dot(xb, wr_ref[...])

    return pl.pallas_call(
        body,
        grid=grid,
        in_specs=[
            pl.BlockSpec((bn, d), lambda i: (i, 0)),
            pl.BlockSpec((d, d), lambda i: (0, 0)),
            pl.BlockSpec((d, d), lambda i: (0, 0)),
            pl.BlockSpec((1, d), lambda i: (0, 0)),
        ],
        out_specs=[pl.BlockSpec((bn, d), lambda i: (i, 0))] * 2,
        out_shape=[jax.ShapeDtypeStruct((n, d), jnp.float32)] * 2,
    )(x, wl, wr, bp.reshape(1, d))


def _combine_mid(p0, p1, c0, c1, h, wfl, wfr, bf, wpl, wpr, bp):
    n, d = h.shape
    bn = 2000
    grid = (n // bn,)

    def body(p0_ref, p1_ref, c0_ref, c1_ref, h_ref, wfl_ref, wfr_ref, bf_ref,
             wpl_ref, wpr_ref, bp_ref, h1_ref, a_ref, b_ref):
        s = p0_ref[...] + p1_ref[...]
        cnt = c0_ref[...] + c1_ref[...]
        agg = s / jnp.maximum(cnt[:, :1], 1.0)
        hb = h_ref[...]
        pre = _dot(hb, wfl_ref[...]) + _dot(agg, wfr_ref[...]) + bf_ref[...]
        h1 = jnp.maximum(pre, 0.0) + hb
        h1_ref[...] = h1
        a_ref[...] = _dot(h1, wpl_ref[...]) + bp_ref[...]
        b_ref[...] = _dot(h1, wpr_ref[...])

    row_spec = pl.BlockSpec((bn, d), lambda i: (i, 0))
    cnt_spec = pl.BlockSpec((bn, d), lambda i: (i, 0))
    w_spec = pl.BlockSpec((d, d), lambda i: (0, 0))
    b_spec = pl.BlockSpec((1, d), lambda i: (0, 0))
    return pl.pallas_call(
        body,
        grid=grid,
        in_specs=[row_spec, row_spec, cnt_spec, cnt_spec, row_spec,
                  w_spec, w_spec, b_spec, w_spec, w_spec, b_spec],
        out_specs=[row_spec] * 3,
        out_shape=[jax.ShapeDtypeStruct((n, d), jnp.float32)] * 3,
    )(p0, p1, c0, c1, h, wfl, wfr, bf.reshape(1, d), wpl, wpr,
      bp.reshape(1, d))


def _combine_final(p0, p1, c0, c1, h, wfl, wfr, bf, wd, bd):
    n, d = h.shape
    bn = 2000
    grid = (n // bn,)

    def body(p0_ref, p1_ref, c0_ref, c1_ref, h_ref, wfl_ref, wfr_ref, bf_ref,
             wd_ref, bd_ref, o_ref):
        s = p0_ref[...] + p1_ref[...]
        cnt = c0_ref[...] + c1_ref[...]
        agg = s / jnp.maximum(cnt[:, :1], 1.0)
        hb = h_ref[...]
        pre = _dot(hb, wfl_ref[...]) + _dot(agg, wfr_ref[...]) + bf_ref[...]
        h2 = jnp.maximum(pre, 0.0) + hb
        o_ref[...] = _dot(h2, wd_ref[...]) + bd_ref[...]

    row_spec = pl.BlockSpec((bn, d), lambda i: (i, 0))
    cnt_spec = pl.BlockSpec((bn, d), lambda i: (i, 0))
    w_spec = pl.BlockSpec((d, d), lambda i: (0, 0))
    b_spec = pl.BlockSpec((1, d), lambda i: (0, 0))
    return pl.pallas_call(
        body,
        grid=grid,
        in_specs=[row_spec, row_spec, cnt_spec, cnt_spec, row_spec,
                  w_spec, w_spec, b_spec, w_spec, b_spec],
        out_specs=row_spec,
        out_shape=jax.ShapeDtypeStruct((n, d), jnp.float32),
    )(p0, p1, c0, c1, h, wfl, wfr, bf.reshape(1, d), wd, bd.reshape(1, d))


def kernel(x, edge_index, W_psi0, b_psi0, W_phi0, b_phi0,
           W_psi1, b_psi1, W_phi1, b_phi1, W_down, b_down):
    n, d = x.shape
    e = edge_index.shape[1]
    src = edge_index[0]
    dst = edge_index[1]

    n_pad = -(-n // (NS * 128)) * (NS * 128)
    zeros = jnp.zeros((n_pad, d), jnp.float32)

    a0, b0 = _psi_pre(x, W_psi0[:, :d].T, W_psi0[:, d:].T, b_psi0)
    edge_pass = _make_edge_kernel(n, d, e)
    count_pass = _make_count_kernel(n, d, e)
    cntp = count_pass(dst, zeros)[:, :n]
    p = edge_pass(a0, b0, src, dst, zeros)[:, :n]

    h1, a1, b1 = _combine_mid(
        p[0], p[1], cntp[0], cntp[1], x,
        W_phi0[:, :d].T, W_phi0[:, d:].T, b_phi0,
        W_psi1[:, :d].T, W_psi1[:, d:].T, b_psi1)

    p2 = edge_pass(a1, b1, src, dst, zeros)[:, :n]

    return _combine_final(
        p2[0], p2[1], cntp[0], cntp[1], h1,
        W_phi1[:, :d].T, W_phi1[:, d:].T, b_phi1,
        W_down.T, b_down)


# speedup vs baseline: 5.2497x; 1.0516x over previous
import jax
import jax.numpy as jnp
from jax import lax
from jax.experimental import pallas as pl
from jax.experimental.pallas import tpu as pltpu
from jax.experimental.pallas import tpu_sc as plsc

NC = 2
NS = 16
NW = NC * NS
LANES = 16

_PREC = jax.lax.Precision.HIGHEST


def _grid_consts(n, e, c):
    epw = e // NW
    nchunk = epw // c
    n_pad = -(-n // (NS * 128)) * (NS * 128)
    rpw = n_pad // NS
    assert epw * NW == e and c * nchunk == epw and c % 8 == 0
    return epw, nchunk, n_pad, rpw


def _sc_mesh():
    return plsc.VectorSubcoreMesh(
        core_axis_name="c", subcore_axis_name="s",
        num_cores=NC, num_subcores=NS)


def _make_edge_kernel(n, d, e):
    c = 40
    epw, nchunk, n_pad, rpw = _grid_consts(n, e, c)
    assert d % LANES == 0
    dv = d // LANES
    nslot = -(-nchunk // 4) * 4
    assert nchunk >= 4

    scratch = (
        [pltpu.VMEM((c,), jnp.int32) for _ in range(4)]
        + [pltpu.VMEM((c,), jnp.int32) for _ in range(4)]
        + [pltpu.VMEM((c, d), jnp.float32) for _ in range(4)]
        + [pltpu.VMEM_SHARED((n_pad, d), jnp.float32)]
        + [pltpu.SemaphoreType.DMA] * 6
    )

    def body(a_hbm, b_hbm, src_hbm, dst_hbm, z_hbm, out_hbm, *refs):
        idx_s = refs[0:4]
        idx_d = refs[4:8]
        ra = refs[8:10]
        rb = refs[10:12]
        acc = refs[12]
        isem = refs[13:17]
        gsem = refs[17:19]

        cid = lax.axis_index("c")
        sid = lax.axis_index("s")
        wid = cid * NS + sid
        rs = pl.ds(sid * rpw, rpw)
        base = wid * epw

        def idx_issue(k, m):
            off = base + k * c
            pltpu.async_copy(src_hbm.at[pl.ds(off, c)], idx_s[m], isem[m])
            pltpu.async_copy(dst_hbm.at[pl.ds(off, c)], idx_d[m], isem[m])

        def idx_wait(m):
            pltpu.make_async_copy(src_hbm.at[pl.ds(0, c)], idx_s[m],
                                  isem[m]).wait()
            pltpu.make_async_copy(dst_hbm.at[pl.ds(0, c)], idx_d[m],
                                  isem[m]).wait()

        def gather_issue(m, p):
            pltpu.async_copy(a_hbm.at[idx_d[m]], ra[p], gsem[p])
            pltpu.async_copy(b_hbm.at[idx_s[m]], rb[p], gsem[p])

        def gather_wait(m, p):
            pltpu.make_async_copy(a_hbm.at[idx_d[m]], ra[p], gsem[p]).wait()
            pltpu.make_async_copy(b_hbm.at[idx_s[m]], rb[p], gsem[p]).wait()

        pltpu.sync_copy(z_hbm.at[rs], acc.at[rs])
        plsc.subcore_barrier()

        for kk in range(3):
            idx_issue(kk, kk)
        idx_wait(0)
        gather_issue(0, 0)

        def outer(g, carry):
            for b in range(4):
                k = g * 4 + b
                p = b % 2
                q = 1 - p
                m = b
                m1 = (b + 1) % 4
                m3 = (b + 3) % 4

                @pl.when(k < nchunk)
                def _():
                    gather_wait(m, p)

                    def row(r, rc):
                        for j in range(dv):
                            sl = pl.ds(j * LANES, LANES)
                            ra[p][r, sl] = jnp.maximum(
                                ra[p][r, sl] + rb[p][r, sl], 0.0)
                        return rc
                    lax.fori_loop(0, c, row, 0)

                @pl.when(k < nchunk)
                def _():
                    pltpu.sync_copy(ra[p], acc.at[idx_d[m]], add=True)

                @pl.when(k + 1 < nchunk)
                def _():
                    idx_wait(m1)
                    gather_issue(m1, q)

                @pl.when(k + 3 < nchunk)
                def _():
                    idx_issue(k + 3, m3)
            return carry
        lax.fori_loop(0, nslot // 4, outer, 0)

        plsc.subcore_barrier()
        pltpu.sync_copy(acc.at[rs], out_hbm.at[cid, rs])

    return pl.kernel(body,
                     out_type=jax.ShapeDtypeStruct((NC, n_pad, d),
                                                   jnp.float32),
                     mesh=_sc_mesh(), scratch_types=scratch)


def _make_count_kernel(n, d, e):
    c = 80
    epw, nchunk, n_pad, rpw = _grid_consts(n, e, c)
    dv = d // LANES
    nslot = -(-nchunk // 4) * 4
    assert nchunk >= 4

    scratch = (
        [pltpu.VMEM((c,), jnp.int32) for _ in range(4)]
        + [pltpu.VMEM((c, d), jnp.float32)]
        + [pltpu.VMEM_SHARED((n_pad, d), jnp.float32)]
        + [pltpu.SemaphoreType.DMA] * 4
    )

    def body(dst_hbm, z_hbm, cnt_hbm, *refs):
        idx_d = refs[0:4]
        ones_v = refs[4]
        cnt_acc = refs[5]
        isem = refs[6:10]

        cid = lax.axis_index("c")
        sid = lax.axis_index("s")
        wid = cid * NS + sid
        one16 = jnp.ones((LANES,), jnp.float32)
        rs = pl.ds(sid * rpw, rpw)
        base = wid * epw

        def idx_issue(k, m):
            pltpu.async_copy(dst_hbm.at[pl.ds(base + k * c, c)], idx_d[m],
                             isem[m])

        def idx_wait(m):
            pltpu.make_async_copy(dst_hbm.at[pl.ds(0, c)], idx_d[m],
                                  isem[m]).wait()

        def orow(r, carry):
            for j in range(dv):
                ones_v[r, pl.ds(j * LANES, LANES)] = one16
            return carry
        lax.fori_loop(0, c, orow, 0)
        pltpu.sync_copy(z_hbm.at[rs], cnt_acc.at[rs])
        plsc.subcore_barrier()

        for kk in range(3):
            idx_issue(kk, kk)

        def outer(g, carry):
            for b in range(4):
                k = g * 4 + b
                m = b
                m3 = (b + 3) % 4

                @pl.when(k < nchunk)
                def _():
                    idx_wait(m)
                    pltpu.sync_copy(ones_v, cnt_acc.at[idx_d[m]], add=True)

                @pl.when(k + 3 < nchunk)
                def _():
                    idx_issue(k + 3, m3)
            return carry
        lax.fori_loop(0, nslot // 4, outer, 0)

        plsc.subcore_barrier()
        pltpu.sync_copy(cnt_acc.at[rs], cnt_hbm.at[cid, rs])

    return pl.kernel(body,
                     out_type=jax.ShapeDtypeStruct((NC, n_pad, d),
                                                   jnp.float32),
                     mesh=_sc_mesh(), scratch_types=scratch)


def _dot(a, b):
    return jnp.dot(a, b, preferred_element_type=jnp.float32, precision=_PREC)


def _psi_pre(x, wl, wr, bp):
    n, d = x.shape
    bn = 2000
    grid = (n // bn,)

    def body(x_ref, wl_ref, wr_ref, bp_ref, a_ref, b_ref):
        xb = x_ref[...]
        a_ref[...] = _dot(xb, wl_ref[...]) + bp_ref[...]
        b_ref[...] = _dot(xb, wr_ref[...])

    return pl.pallas_call(
        body,
        grid=grid,
        in_specs=[
            pl.BlockSpec((bn, d), lambda i: (i, 0)),
            pl.BlockSpec((d, d), lambda i: (0, 0)),
            pl.BlockSpec((d, d), lambda i: (0, 0)),
            pl.BlockSpec((1, d), lambda i: (0, 0)),
        ],
        out_specs=[pl.BlockSpec((bn, d), lambda i: (i, 0))] * 2,
        out_shape=[jax.ShapeDtypeStruct((n, d), jnp.float32)] * 2,
    )(x, wl, wr, bp.reshape(1, d))


def _combine_mid(p0, p1, c0, c1, h, wfl, wfr, bf, wpl, wpr, bp):
    n, d = h.shape
    bn = 2000
    grid = (n // bn,)

    def body(p0_ref, p1_ref, c0_ref, c1_ref, h_ref, wfl_ref, wfr_ref, bf_ref,
             wpl_ref, wpr_ref, bp_ref, h1_ref, a_ref, b_ref):
        s = p0_ref[...] + p1_ref[...]
        cnt = c0_ref[...] + c1_ref[...]
        agg = s / jnp.maximum(cnt[:, :1], 1.0)
        hb = h_ref[...]
        pre = _dot(hb, wfl_ref[...]) + _dot(agg, wfr_ref[...]) + bf_ref[...]
        h1 = jnp.maximum(pre, 0.0) + hb
        h1_ref[...] = h1
        a_ref[...] = _dot(h1, wpl_ref[...]) + bp_ref[...]
        b_ref[...] = _dot(h1, wpr_ref[...])

    row_spec = pl.BlockSpec((bn, d), lambda i: (i, 0))
    cnt_spec = pl.BlockSpec((bn, d), lambda i: (i, 0))
    w_spec = pl.BlockSpec((d, d), lambda i: (0, 0))
    b_spec = pl.BlockSpec((1, d), lambda i: (0, 0))
    return pl.pallas_call(
        body,
        grid=grid,
        in_specs=[row_spec, row_spec, cnt_spec, cnt_spec, row_spec,
                  w_spec, w_spec, b_spec, w_spec, w_spec, b_spec],
        out_specs=[row_spec] * 3,
        out_shape=[jax.ShapeDtypeStruct((n, d), jnp.float32)] * 3,
    )(p0, p1, c0, c1, h, wfl, wfr, bf.reshape(1, d), wpl, wpr,
      bp.reshape(1, d))


def _combine_final(p0, p1, c0, c1, h, wfl, wfr, bf, wd, bd):
    n, d = h.shape
    bn = 2000
    grid = (n // bn,)

    def body(p0_ref, p1_ref, c0_ref, c1_ref, h_ref, wfl_ref, wfr_ref, bf_ref,
             wd_ref, bd_ref, o_ref):
        s = p0_ref[...] + p1_ref[...]
        cnt = c0_ref[...] + c1_ref[...]
        agg = s / jnp.maximum(cnt[:, :1], 1.0)
        hb = h_ref[...]
        pre = _dot(hb, wfl_ref[...]) + _dot(agg, wfr_ref[...]) + bf_ref[...]
        h2 = jnp.maximum(pre, 0.0) + hb
        o_ref[...] = _dot(h2, wd_ref[...]) + bd_ref[...]

    row_spec = pl.BlockSpec((bn, d), lambda i: (i, 0))
    cnt_spec = pl.BlockSpec((bn, d), lambda i: (i, 0))
    w_spec = pl.BlockSpec((d, d), lambda i: (0, 0))
    b_spec = pl.BlockSpec((1, d), lambda i: (0, 0))
    return pl.pallas_call(
        body,
        grid=grid,
        in_specs=[row_spec, row_spec, cnt_spec, cnt_spec, row_spec,
                  w_spec, w_spec, b_spec, w_spec, b_spec],
        out_specs=row_spec,
        out_shape=jax.ShapeDtypeStruct((n, d), jnp.float32),
    )(p0, p1, c0, c1, h, wfl, wfr, bf.reshape(1, d), wd, bd.reshape(1, d))


def kernel(x, edge_index, W_psi0, b_psi0, W_phi0, b_phi0,
           W_psi1, b_psi1, W_phi1, b_phi1, W_down, b_down):
    n, d = x.shape
    e = edge_index.shape[1]
    src = edge_index[0]
    dst = edge_index[1]

    n_pad = -(-n // (NS * 128)) * (NS * 128)
    zeros = jnp.zeros((n_pad, d), jnp.float32)

    a0, b0 = _psi_pre(x, W_psi0[:, :d].T, W_psi0[:, d:].T, b_psi0)
    edge_pass = _make_edge_kernel(n, d, e)
    count_pass = _make_count_kernel(n, d, e)
    cntp = count_pass(dst, zeros)[:, :n]
    p = edge_pass(a0, b0, src, dst, zeros)[:, :n]

    h1, a1, b1 = _combine_mid(
        p[0], p[1], cntp[0], cntp[1], x,
        W_phi0[:, :d].T, W_phi0[:, d:].T, b_phi0,
        W_psi1[:, :d].T, W_psi1[:, d:].T, b_psi1)

    p2 = edge_pass(a1, b1, src, dst, zeros)[:, :n]

    return _combine_final(
        p2[0], p2[1], cntp[0], cntp[1], h1,
        W_phi1[:, :d].T, W_phi1[:, d:].T, b_phi1,
        W_down.T, b_down)


# speedup vs baseline: 7.2038x; 1.3722x over previous
import jax
import jax.numpy as jnp
from jax import lax
from jax.experimental import pallas as pl
from jax.experimental.pallas import tpu as pltpu
from jax.experimental.pallas import tpu_sc as plsc

NC = 2
NS = 16
NW = NC * NS
LANES = 16

_PREC = jax.lax.Precision.HIGHEST


def _grid_consts(n, e, c):
    epw = e // NW
    nchunk = epw // c
    n_pad = -(-n // (NS * 128)) * (NS * 128)
    rpw = n_pad // NS
    assert epw * NW == e and c * nchunk == epw and c % 8 == 0
    return epw, nchunk, n_pad, rpw


def _sc_mesh():
    return plsc.VectorSubcoreMesh(
        core_axis_name="c", subcore_axis_name="s",
        num_cores=NC, num_subcores=NS)


def _make_edge_kernel(n, d, e):
    c = 40
    epw, nchunk, n_pad, rpw = _grid_consts(n, e, c)
    assert d % LANES == 0
    dv = d // LANES
    nslot = -(-nchunk // 4) * 4
    assert nchunk >= 4

    scratch = (
        [pltpu.VMEM((c,), jnp.int32) for _ in range(4)]
        + [pltpu.VMEM((c,), jnp.int32) for _ in range(4)]
        + [pltpu.VMEM((c, d), jnp.float32) for _ in range(4)]
        + [pltpu.VMEM_SHARED((n_pad, d), jnp.float32)]
        + [pltpu.SemaphoreType.DMA] * 6
    )

    def body(a_hbm, b_hbm, src_hbm, dst_hbm, z_hbm, out_hbm, *refs):
        idx_s = refs[0:4]
        idx_d = refs[4:8]
        ra = refs[8:10]
        rb = refs[10:12]
        acc = refs[12]
        isem = refs[13:17]
        gsem = refs[17:19]

        cid = lax.axis_index("c")
        sid = lax.axis_index("s")
        wid = cid * NS + sid
        rs = pl.ds(sid * rpw, rpw)
        base = wid * epw

        def idx_issue(k, m):
            off = base + k * c
            pltpu.async_copy(src_hbm.at[pl.ds(off, c)], idx_s[m], isem[m])
            pltpu.async_copy(dst_hbm.at[pl.ds(off, c)], idx_d[m], isem[m])

        def idx_wait(m):
            pltpu.make_async_copy(src_hbm.at[pl.ds(0, c)], idx_s[m],
                                  isem[m]).wait()
            pltpu.make_async_copy(dst_hbm.at[pl.ds(0, c)], idx_d[m],
                                  isem[m]).wait()

        def gather_issue(m, p):
            pltpu.async_copy(a_hbm.at[idx_d[m]], ra[p], gsem[p])
            pltpu.async_copy(b_hbm.at[idx_s[m]], rb[p], gsem[p])

        def gather_wait(m, p):
            pltpu.make_async_copy(a_hbm.at[idx_d[m]], ra[p], gsem[p]).wait()
            pltpu.make_async_copy(b_hbm.at[idx_s[m]], rb[p], gsem[p]).wait()

        pltpu.sync_copy(z_hbm.at[rs], acc.at[rs])
        plsc.subcore_barrier()

        for kk in range(3):
            idx_issue(kk, kk)
        idx_wait(0)
        gather_issue(0, 0)

        def outer(g, carry):
            for b in range(4):
                k = g * 4 + b
                p = b % 2
                q = 1 - p
                m = b
                m1 = (b + 1) % 4
                m3 = (b + 3) % 4

                @pl.when(k < nchunk)
                def _():
                    gather_wait(m, p)

                @pl.when(k + 1 < nchunk)
                def _():
                    idx_wait(m1)
                    gather_issue(m1, q)

                @pl.when(k + 3 < nchunk)
                def _():
                    idx_issue(k + 3, m3)

                @pl.when(k < nchunk)
                def _():
                    def row(r, rc):
                        for j in range(dv):
                            sl = pl.ds(j * LANES, LANES)
                            ra[p][r, sl] = jnp.maximum(
                                ra[p][r, sl] + rb[p][r, sl], 0.0)
                        return rc
                    lax.fori_loop(0, c, row, 0)
                    pltpu.sync_copy(ra[p], acc.at[idx_d[m]], add=True)
            return carry
        lax.fori_loop(0, nslot // 4, outer, 0)

        plsc.subcore_barrier()
        pltpu.sync_copy(acc.at[rs], out_hbm.at[cid, rs])

    return pl.kernel(body,
                     out_type=jax.ShapeDtypeStruct((NC, n_pad, d),
                                                   jnp.float32),
                     mesh=_sc_mesh(), scratch_types=scratch)


def _make_count_kernel(n, d, e):
    c = 80
    epw, nchunk, n_pad, rpw = _grid_consts(n, e, c)
    dv = d // LANES
    nslot = -(-nchunk // 4) * 4
    assert nchunk >= 4

    scratch = (
        [pltpu.VMEM((c,), jnp.int32) for _ in range(4)]
        + [pltpu.VMEM((c, d), jnp.float32)]
        + [pltpu.VMEM_SHARED((n_pad, d), jnp.float32)]
        + [pltpu.SemaphoreType.DMA] * 4
    )

    def body(dst_hbm, z_hbm, cnt_hbm, *refs):
        idx_d = refs[0:4]
        ones_v = refs[4]
        cnt_acc = refs[5]
        isem = refs[6:10]

        cid = lax.axis_index("c")
        sid = lax.axis_index("s")
        wid = cid * NS + sid
        one16 = jnp.ones((LANES,), jnp.float32)
        rs = pl.ds(sid * rpw, rpw)
        base = wid * epw

        def idx_issue(k, m):
            pltpu.async_copy(dst_hbm.at[pl.ds(base + k * c, c)], idx_d[m],
                             isem[m])

        def idx_wait(m):
            pltpu.make_async_copy(dst_hbm.at[pl.ds(0, c)], idx_d[m],
                                  isem[m]).wait()

        def orow(r, carry):
            for j in range(dv):
                ones_v[r, pl.ds(j * LANES, LANES)] = one16
            return carry
        lax.fori_loop(0, c, orow, 0)
        pltpu.sync_copy(z_hbm.at[rs], cnt_acc.at[rs])
        plsc.subcore_barrier()

        for kk in range(3):
            idx_issue(kk, kk)

        def outer(g, carry):
            for b in range(4):
                k = g * 4 + b
                m = b
                m3 = (b + 3) % 4

                @pl.when(k < nchunk)
                def _():
                    idx_wait(m)
                    pltpu.sync_copy(ones_v, cnt_acc.at[idx_d[m]], add=True)

                @pl.when(k + 3 < nchunk)
                def _():
                    idx_issue(k + 3, m3)
            return carry
        lax.fori_loop(0, nslot // 4, outer, 0)

        plsc.subcore_barrier()
        pltpu.sync_copy(cnt_acc.at[rs], cnt_hbm.at[cid, rs])

    return pl.kernel(body,
                     out_type=jax.ShapeDtypeStruct((NC, n_pad, d),
                                                   jnp.float32),
                     mesh=_sc_mesh(), scratch_types=scratch)


def _dot(a, b):
    return jnp.dot(a, b, preferred_element_type=jnp.float32, precision=_PREC)


def _psi_pre(x, wl, wr, bp):
    n, d = x.shape
    bn = 2000
    grid = (n // bn,)

    def body(x_ref, wl_ref, wr_ref, bp_ref, a_ref, b_ref):
        xb = x_ref[...]
        a_ref[...] = _dot(xb, wl_ref[...]) + bp_ref[...]
        b_ref[...] = _dot(xb, wr_ref[...])

    return pl.pallas_call(
        body,
        grid=grid,
        in_specs=[
            pl.BlockSpec((bn, d), lambda i: (i, 0)),
            pl.BlockSpec((d, d), lambda i: (0, 0)),
            pl.BlockSpec((d, d), lambda i: (0, 0)),
            pl.BlockSpec((1, d), lambda i: (0, 0)),
        ],
        out_specs=[pl.BlockSpec((bn, d), lambda i: (i, 0))] * 2,
        out_shape=[jax.ShapeDtypeStruct((n, d), jnp.float32)] * 2,
    )(x, wl, wr, bp.reshape(1, d))


def _combine_mid(p0, p1, c0, c1, h, wfl, wfr, bf, wpl, wpr, bp):
    n, d = h.shape
    bn = 2000
    grid = (n // bn,)

    def body(p0_ref, p1_ref, c0_ref, c1_ref, h_ref, wfl_ref, wfr_ref, bf_ref,
             wpl_ref, wpr_ref, bp_ref, h1_ref, a_ref, b_ref):
        s = p0_ref[...] + p1_ref[...]
        cnt = c0_ref[...] + c1_ref[...]
        agg = s / jnp.maximum(cnt[:, :1], 1.0)
        hb = h_ref[...]
        pre = _dot(hb, wfl_ref[...]) + _dot(agg, wfr_ref[...]) + bf_ref[...]
        h1 = jnp.maximum(pre, 0.0) + hb
        h1_ref[...] = h1
        a_ref[...] = _dot(h1, wpl_ref[...]) + bp_ref[...]
        b_ref[...] = _dot(h1, wpr_ref[...])

    row_spec = pl.BlockSpec((bn, d), lambda i: (i, 0))
    cnt_spec = pl.BlockSpec((bn, d), lambda i: (i, 0))
    w_spec = pl.BlockSpec((d, d), lambda i: (0, 0))
    b_spec = pl.BlockSpec((1, d), lambda i: (0, 0))
    return pl.pallas_call(
        body,
        grid=grid,
        in_specs=[row_spec, row_spec, cnt_spec, cnt_spec, row_spec,
                  w_spec, w_spec, b_spec, w_spec, w_spec, b_spec],
        out_specs=[row_spec] * 3,
        out_shape=[jax.ShapeDtypeStruct((n, d), jnp.float32)] * 3,
    )(p0, p1, c0, c1, h, wfl, wfr, bf.reshape(1, d), wpl, wpr,
      bp.reshape(1, d))


def _combine_final(p0, p1, c0, c1, h, wfl, wfr, bf, wd, bd):
    n, d = h.shape
    bn = 2000
    grid = (n // bn,)

    def body(p0_ref, p1_ref, c0_ref, c1_ref, h_ref, wfl_ref, wfr_ref, bf_ref,
             wd_ref, bd_ref, o_ref):
        s = p0_ref[...] + p1_ref[...]
        cnt = c0_ref[...] + c1_ref[...]
        agg = s / jnp.maximum(cnt[:, :1], 1.0)
        hb = h_ref[...]
        pre = _dot(hb, wfl_ref[...]) + _dot(agg, wfr_ref[...]) + bf_ref[...]
        h2 = jnp.maximum(pre, 0.0) + hb
        o_ref[...] = _dot(h2, wd_ref[...]) + bd_ref[...]

    row_spec = pl.BlockSpec((bn, d), lambda i: (i, 0))
    cnt_spec = pl.BlockSpec((bn, d), lambda i: (i, 0))
    w_spec = pl.BlockSpec((d, d), lambda i: (0, 0))
    b_spec = pl.BlockSpec((1, d), lambda i: (0, 0))
    return pl.pallas_call(
        body,
        grid=grid,
        in_specs=[row_spec, row_spec, cnt_spec, cnt_spec, row_spec,
                  w_spec, w_spec, b_spec, w_spec, b_spec],
        out_specs=row_spec,
        out_shape=jax.ShapeDtypeStruct((n, d), jnp.float32),
    )(p0, p1, c0, c1, h, wfl, wfr, bf.reshape(1, d), wd, bd.reshape(1, d))


def kernel(x, edge_index, W_psi0, b_psi0, W_phi0, b_phi0,
           W_psi1, b_psi1, W_phi1, b_phi1, W_down, b_down):
    n, d = x.shape
    e = edge_index.shape[1]
    src = edge_index[0]
    dst = edge_index[1]

    n_pad = -(-n // (NS * 128)) * (NS * 128)
    zeros = jnp.zeros((n_pad, d), jnp.float32)

    a0, b0 = _psi_pre(x, W_psi0[:, :d].T, W_psi0[:, d:].T, b_psi0)
    edge_pass = _make_edge_kernel(n, d, e)
    count_pass = _make_count_kernel(n, d, e)
    cntp = count_pass(dst, zeros)[:, :n]
    p = edge_pass(a0, b0, src, dst, zeros)[:, :n]

    h1, a1, b1 = _combine_mid(
        p[0], p[1], cntp[0], cntp[1], x,
        W_phi0[:, :d].T, W_phi0[:, d:].T, b_phi0,
        W_psi1[:, :d].T, W_psi1[:, d:].T, b_psi1)

    p2 = edge_pass(a1, b1, src, dst, zeros)[:, :n]

    return _combine_final(
        p2[0], p2[1], cntp[0], cntp[1], h1,
        W_phi1[:, :d].T, W_phi1[:, d:].T, b_phi1,
        W_down.T, b_down)
